# Initial kernel scaffold; baseline (speedup 1.0000x reference)
import functools

import jax
import jax.numpy as jnp
from jax import lax
from jax.experimental import pallas as pl
from jax.experimental.pallas import tpu as pltpu
from jax.experimental.pallas import tpu_sc as plsc

L = 16
NUM_CORES = 2
NUM_SUBCORES = 16
NW = NUM_CORES * NUM_SUBCORES
CHUNK = 8192


def kernel(positions, amplitudes, hermite_basis):
    B, S = positions.shape
    N, R = hermite_basis.shape
    rows_per_w = B // NW
    n_chunks = S // CHUNK

    pos_flat = positions.reshape(-1)
    amp_flat = amplitudes.reshape(-1)
    bas_flat = hermite_basis.reshape(-1)

    mesh = plsc.VectorSubcoreMesh(core_axis_name="c", subcore_axis_name="s")

    @functools.partial(
        pl.kernel,
        out_type=jax.ShapeDtypeStruct((B * S,), jnp.float32),
        mesh=mesh,
        scratch_types=[
            pltpu.VMEM((rows_per_w * N,), jnp.float32),
            pltpu.VMEM((N * R,), jnp.float32),
            pltpu.VMEM((R,), jnp.float32),
            pltpu.VMEM((CHUNK,), jnp.float32),
            pltpu.VMEM((CHUNK,), jnp.float32),
        ],
    )
    def _k(pos_hbm, amp_hbm, bas_hbm, out_hbm, amp_v, bas_v, comb_v, pos_v, out_v):
        wid = lax.axis_index("s") * NUM_CORES + lax.axis_index("c")
        pltpu.sync_copy(bas_hbm, bas_v)
        pltpu.sync_copy(
            amp_hbm.at[pl.ds(wid * (rows_per_w * N), rows_per_w * N)], amp_v
        )

        for b in range(rows_per_w):
            amps = [
                plsc.load_gather(amp_v, [jnp.full((L,), b * N + n, jnp.int32)])
                for n in range(N)
            ]
            for rc in range(R // L):
                acc = amps[0] * bas_v[pl.ds(rc * L, L)]
                for n in range(1, N):
                    acc = acc + amps[n] * bas_v[pl.ds(n * R + rc * L, L)]
                comb_v[pl.ds(rc * L, L)] = acc

            row = wid * rows_per_w + b
            for c in range(n_chunks):
                base = pl.multiple_of(row * S + c * CHUNK, CHUNK)
                pltpu.sync_copy(pos_hbm.at[pl.ds(base, CHUNK)], pos_v)

                def body(i, _):
                    p = pos_v[pl.ds(i * L, L)]
                    f = (p + 1.0) * (0.5 * (R - 1))
                    idx = f.astype(jnp.int32)
                    idx = jnp.minimum(jnp.maximum(idx, 0), R - 1)
                    out_v[pl.ds(i * L, L)] = plsc.load_gather(comb_v, [idx])
                    return 0

                lax.fori_loop(0, CHUNK // L, body, 0)
                pltpu.sync_copy(out_v, out_hbm.at[pl.ds(base, CHUNK)])

    out = _k(pos_flat, amp_flat, bas_flat)
    return out.reshape(B, S)


# speedup vs baseline: 127.4349x; 127.4349x over previous
import functools

import jax
import jax.numpy as jnp
from jax import lax
from jax.experimental import pallas as pl
from jax.experimental.pallas import tpu as pltpu
from jax.experimental.pallas import tpu_sc as plsc

L = 16
NUM_CORES = 2
NUM_SUBCORES = 16
NW = NUM_CORES * NUM_SUBCORES
CHUNK = 8192


def kernel(positions, amplitudes, hermite_basis):
    B, S = positions.shape
    N, R = hermite_basis.shape
    rows_per_w = B // NW
    n_chunks = S // CHUNK

    pos_flat = positions.reshape(-1)
    amp_bcast = jnp.broadcast_to(
        amplitudes.reshape(B, N, 1), (B, N, L)
    ).reshape(-1)
    bas_flat = hermite_basis.reshape(-1)

    mesh = plsc.VectorSubcoreMesh(
        core_axis_name="c", subcore_axis_name="s",
        num_cores=NUM_CORES, num_subcores=NUM_SUBCORES,
    )

    @functools.partial(
        pl.kernel,
        out_type=jax.ShapeDtypeStruct((B * S,), jnp.float32),
        mesh=mesh,
        compiler_params=pltpu.CompilerParams(needs_layout_passes=False),
        scratch_types=[
            pltpu.VMEM((rows_per_w * N * L,), jnp.float32),
            pltpu.VMEM((N * R,), jnp.float32),
            pltpu.VMEM((rows_per_w * R,), jnp.float32),
            pltpu.VMEM((CHUNK,), jnp.float32),
            pltpu.VMEM((CHUNK,), jnp.float32),
        ],
    )
    def _k(pos_hbm, amp_hbm, bas_hbm, out_hbm, amp_v, bas_v, comb_v, pos_v, out_v):
        wid = lax.axis_index("s") * NUM_CORES + lax.axis_index("c")
        pltpu.sync_copy(bas_hbm, bas_v)
        pltpu.sync_copy(
            amp_hbm.at[pl.ds(wid * (rows_per_w * N * L), rows_per_w * N * L)],
            amp_v,
        )

        for b in range(rows_per_w):
            for rc in range(R // L):
                acc = amp_v[pl.ds(b * N * L, L)] * bas_v[pl.ds(rc * L, L)]
                for n in range(1, N):
                    acc = acc + (
                        amp_v[pl.ds((b * N + n) * L, L)]
                        * bas_v[pl.ds(n * R + rc * L, L)]
                    )
                comb_v[pl.ds(b * R + rc * L, L)] = acc

        for b in range(rows_per_w):
            row = wid * rows_per_w + b
            for c in range(n_chunks):
                base = pl.multiple_of(row * S + c * CHUNK, CHUNK)
                pltpu.sync_copy(pos_hbm.at[pl.ds(base, CHUNK)], pos_v)

                def body(i, _):
                    p = pos_v[pl.ds(i * L, L)]
                    f = (p + 1.0) / 2.0 * (R - 1)
                    idx = f.astype(jnp.int32)
                    idx = jnp.minimum(jnp.maximum(idx, 0), R - 1) + (b * R)
                    out_v[pl.ds(i * L, L)] = plsc.load_gather(comb_v, [idx])
                    return 0

                lax.fori_loop(0, CHUNK // L, body, 0)
                pltpu.sync_copy(out_v, out_hbm.at[pl.ds(base, CHUNK)])

    out = _k(pos_flat, amp_bcast, bas_flat)
    return out.reshape(B, S)


# speedup vs baseline: 225.9608x; 1.7731x over previous
import functools

import jax
import jax.numpy as jnp
from jax import lax
from jax.experimental import pallas as pl
from jax.experimental.pallas import tpu as pltpu
from jax.experimental.pallas import tpu_sc as plsc

L = 16
NUM_CORES = 2
NUM_SUBCORES = 16
NW = NUM_CORES * NUM_SUBCORES
CHUNK = 16384
UNROLL = 8


def kernel(positions, amplitudes, hermite_basis):
    B, S = positions.shape
    N, R = hermite_basis.shape
    rows_per_w = B // NW
    n_chunks = S // CHUNK

    amp_bcast = jnp.broadcast_to(
        amplitudes.reshape(B, N, 1), (B, N, L)
    ).reshape(-1)
    bas_flat = hermite_basis.reshape(-1)

    mesh = plsc.VectorSubcoreMesh(
        core_axis_name="c", subcore_axis_name="s",
        num_cores=NUM_CORES, num_subcores=NUM_SUBCORES,
    )

    @functools.partial(
        pl.kernel,
        out_type=jax.ShapeDtypeStruct((B, S), jnp.float32),
        mesh=mesh,
        compiler_params=pltpu.CompilerParams(needs_layout_passes=False),
        scratch_types=[
            pltpu.VMEM((rows_per_w * N * L,), jnp.float32),
            pltpu.VMEM((N * R,), jnp.float32),
            pltpu.VMEM((rows_per_w * R,), jnp.float32),
            pltpu.VMEM((CHUNK,), jnp.float32),
            pltpu.VMEM((CHUNK,), jnp.float32),
            pltpu.VMEM((CHUNK,), jnp.float32),
            pltpu.VMEM((CHUNK,), jnp.float32),
            pltpu.SemaphoreType.DMA,
            pltpu.SemaphoreType.DMA,
            pltpu.SemaphoreType.DMA,
            pltpu.SemaphoreType.DMA,
        ],
    )
    def _k(pos_hbm, amp_hbm, bas_hbm, out_hbm, amp_v, bas_v, comb_v,
           pos_v0, pos_v1, out_v0, out_v1, isem0, isem1, osem0, osem1):
        pos_bufs = (pos_v0, pos_v1)
        out_bufs = (out_v0, out_v1)
        in_sems = (isem0, isem1)
        out_sems = (osem0, osem1)
        wid = lax.axis_index("s") * NUM_CORES + lax.axis_index("c")
        pltpu.sync_copy(bas_hbm, bas_v)
        pltpu.sync_copy(
            amp_hbm.at[pl.ds(wid * (rows_per_w * N * L), rows_per_w * N * L)],
            amp_v,
        )

        for b in range(rows_per_w):
            for rc in range(R // L):
                acc = amp_v[pl.ds(b * N * L, L)] * bas_v[pl.ds(rc * L, L)]
                for n in range(1, N):
                    acc = acc + (
                        amp_v[pl.ds((b * N + n) * L, L)]
                        * bas_v[pl.ds(n * R + rc * L, L)]
                    )
                comb_v[pl.ds(b * R + rc * L, L)] = acc

        tiles = [(b, c) for b in range(rows_per_w) for c in range(n_chunks)]
        T = len(tiles)

        def src(t):
            b, c = tiles[t]
            row = wid * rows_per_w + b
            return pos_hbm.at[row, pl.ds(pl.multiple_of(c * CHUNK, CHUNK), CHUNK)]

        def dst(t):
            b, c = tiles[t]
            row = wid * rows_per_w + b
            return out_hbm.at[row, pl.ds(pl.multiple_of(c * CHUNK, CHUNK), CHUNK)]

        in_descs = {}
        out_descs = {}
        in_descs[0] = pltpu.async_copy(src(0), pos_bufs[0], in_sems[0])
        for t in range(T):
            b, _c = tiles[t]
            if t + 1 < T:
                in_descs[t + 1] = pltpu.async_copy(
                    src(t + 1), pos_bufs[(t + 1) % 2], in_sems[(t + 1) % 2]
                )
            in_descs[t].wait()
            if t >= 2:
                out_descs[t - 2].wait()
            pos_v = pos_bufs[t % 2]
            out_v = out_bufs[t % 2]

            def body(i, _):
                for u in range(UNROLL):
                    off = i * (L * UNROLL) + u * L
                    p = pos_v[pl.ds(off, L)]
                    f = (p + 1.0) / 2.0 * (R - 1)
                    idx = f.astype(jnp.int32)
                    idx = jnp.minimum(jnp.maximum(idx, 0), R - 1) + (b * R)
                    out_v[pl.ds(off, L)] = plsc.load_gather(comb_v, [idx])
                return 0

            lax.fori_loop(0, CHUNK // (L * UNROLL), body, 0)
            out_descs[t] = pltpu.async_copy(out_v, dst(t), out_sems[t % 2])
        out_descs[T - 2].wait()
        out_descs[T - 1].wait()

    return _k(positions, amp_bcast, bas_flat)
